# Initial kernel scaffold; baseline (speedup 1.0000x reference)
import functools

import jax
import jax.numpy as jnp
from jax import lax
from jax.experimental import pallas as pl
from jax.experimental.pallas import tpu as pltpu
from jax.experimental.pallas import tpu_sc as plsc

N = 10000
D = 128
DOUT = 40
E = 320000

NC = 2
NS = 16
NW = NC * NS
CK = 128
CH = 80
TPW = CH * CK
E_PAD = NW * TPW
NPAD = 10240
RPT = NPAD // NS
RDP = N // NS
ZR = 80

BLK = 1000
NBLK = N // BLK
EPS = 1e-5

_mesh = plsc.VectorSubcoreMesh(core_axis_name="c", subcore_axis_name="s")


@functools.partial(
    pl.kernel,
    out_type=jax.ShapeDtypeStruct((NW, NPAD), jnp.float32),
    mesh=_mesh,
    scratch_types=[
        pltpu.VMEM((TPW,), jnp.int32),
        pltpu.VMEM((NPAD,), jnp.float32),
    ],
)
def _sc_hist(dst_hbm, out_hbm, idxv, hist):
    c = lax.axis_index("c")
    s = lax.axis_index("s")
    wid = s * NC + c
    pltpu.sync_copy(dst_hbm.at[wid], idxv)
    zero = jnp.zeros((16,), jnp.float32)

    @pl.loop(0, NPAD, step=16)
    def _(i):
        hist[pl.ds(i, 16)] = zero

    ones = jnp.ones((16,), jnp.float32)

    @pl.loop(0, TPW, step=16)
    def _(i):
        idx = idxv[pl.ds(i, 16)]
        plsc.addupdate_scatter(hist, [idx], ones)

    pltpu.sync_copy(hist, out_hbm.at[wid])


@functools.partial(
    pl.kernel,
    out_type=jax.ShapeDtypeStruct((NC, N, D), jnp.float32),
    mesh=_mesh,
    scratch_types=[
        pltpu.VMEM((CH, CK), jnp.int32),
        pltpu.VMEM((CH, CK), jnp.int32),
        pltpu.VMEM((CK, D), jnp.float32),
        pltpu.VMEM((CK, D), jnp.float32),
        pltpu.VMEM((ZR, D), jnp.float32),
        pltpu.VMEM_SHARED((NPAD, D), jnp.float32),
        pltpu.SemaphoreType.DMA,
        pltpu.SemaphoreType.DMA,
    ],
)
def _sc_agg(z_hbm, src_hbm, dst_hbm, zer_hbm, out_hbm,
            srcv, dstv, rows_a, rows_b, zbuf, acc, sem_a, sem_b):
    c = lax.axis_index("c")
    s = lax.axis_index("s")
    wid = s * NC + c
    pltpu.sync_copy(src_hbm.at[wid], srcv)
    pltpu.sync_copy(dst_hbm.at[wid], dstv)
    pltpu.sync_copy(zer_hbm, zbuf)

    @pl.loop(0, RPT, step=ZR)
    def _(i):
        pltpu.sync_copy(zbuf, acc.at[pl.ds(s * RPT + i, ZR)])

    plsc.subcore_barrier()

    pltpu.make_async_copy(z_hbm.at[srcv.at[0]], rows_a, sem_a).start()

    @pl.loop(0, CH, step=2)
    def _(j):
        pltpu.make_async_copy(z_hbm.at[srcv.at[j]], rows_a, sem_a).wait()
        pltpu.make_async_copy(z_hbm.at[srcv.at[j + 1]], rows_b, sem_b).start()
        pltpu.sync_copy(rows_a, acc.at[dstv.at[j]], add=True)
        pltpu.make_async_copy(z_hbm.at[srcv.at[j + 1]], rows_b, sem_b).wait()

        @pl.when(j + 2 < CH)
        def _():
            pltpu.make_async_copy(z_hbm.at[srcv.at[j + 2]], rows_a, sem_a).start()

        pltpu.sync_copy(rows_b, acc.at[dstv.at[j + 1]], add=True)

    plsc.subcore_barrier()
    pltpu.sync_copy(acc.at[pl.ds(s * RDP, RDP)], out_hbm.at[c, pl.ds(s * RDP, RDP)])


def _mm_body(x_ref, w_ref, o_ref):
    o_ref[...] = jnp.dot(x_ref[...], w_ref[...],
                         preferred_element_type=jnp.float32,
                         precision=lax.Precision.HIGHEST)


def _tc_mm(x, w):
    m, k = x.shape
    n = w.shape[1]
    return pl.pallas_call(
        _mm_body,
        grid=(m // BLK,),
        in_specs=[pl.BlockSpec((BLK, k), lambda i: (i, 0)),
                  pl.BlockSpec((k, n), lambda i: (0, 0))],
        out_specs=pl.BlockSpec((BLK, n), lambda i: (i, 0)),
        out_shape=jax.ShapeDtypeStruct((m, n), jnp.float32),
    )(x, w)


def _prep_body(hp_ref, o_ref):
    o_ref[...] = lax.rsqrt(1.0 + jnp.sum(hp_ref[...], axis=0, keepdims=True))


def _tc_prep(hp):
    return pl.pallas_call(
        _prep_body,
        out_shape=jax.ShapeDtypeStruct((1, NPAD), jnp.float32),
    )(hp)


def _scale_body(d_ref, x_ref, o_ref):
    o_ref[...] = d_ref[...] * x_ref[...]


def _tc_scale(d, x):
    return pl.pallas_call(
        _scale_body,
        grid=(NBLK,),
        in_specs=[pl.BlockSpec((BLK, 1), lambda i: (i, 0)),
                  pl.BlockSpec((BLK, D), lambda i: (i, 0))],
        out_specs=pl.BlockSpec((BLK, D), lambda i: (i, 0)),
        out_shape=jax.ShapeDtypeStruct((N, D), jnp.float32),
    )(d, x)


def _postagg_body(a_ref, z_ref, d_ref, b_ref, p_ref, s_ref, q_ref):
    agg = a_ref[0] + a_ref[1]
    p = d_ref[...] * (agg + z_ref[...]) + b_ref[...]
    p_ref[...] = p

    @pl.when(pl.program_id(0) == 0)
    def _():
        s_ref[...] = jnp.zeros_like(s_ref)
        q_ref[...] = jnp.zeros_like(q_ref)

    s_ref[...] += jnp.sum(p, axis=0, keepdims=True)
    q_ref[...] += jnp.sum(p * p, axis=0, keepdims=True)


def _tc_postagg(a, z, d, b):
    return pl.pallas_call(
        _postagg_body,
        grid=(NBLK,),
        in_specs=[pl.BlockSpec((2, BLK, D), lambda i: (0, i, 0)),
                  pl.BlockSpec((BLK, D), lambda i: (i, 0)),
                  pl.BlockSpec((BLK, 1), lambda i: (i, 0)),
                  pl.BlockSpec((1, D), lambda i: (0, 0))],
        out_specs=[pl.BlockSpec((BLK, D), lambda i: (i, 0)),
                   pl.BlockSpec((1, D), lambda i: (0, 0)),
                   pl.BlockSpec((1, D), lambda i: (0, 0))],
        out_shape=[jax.ShapeDtypeStruct((N, D), jnp.float32),
                   jax.ShapeDtypeStruct((1, D), jnp.float32),
                   jax.ShapeDtypeStruct((1, D), jnp.float32)],
    )(a, z, d, b)


def _bn_core(p_ref, s_ref, q_ref, g_ref, be_ref):
    mu = s_ref[...] * (1.0 / N)
    var = q_ref[...] * (1.0 / N) - mu * mu
    a = g_ref[...] * lax.rsqrt(var + EPS)
    return jnp.maximum((p_ref[...] - mu) * a + be_ref[...], 0.0)


def _bnmm_body(p_ref, s_ref, q_ref, g_ref, be_ref, d_ref, w_ref, o_ref):
    h = _bn_core(p_ref, s_ref, q_ref, g_ref, be_ref)
    o_ref[...] = d_ref[...] * jnp.dot(h, w_ref[...],
                                      preferred_element_type=jnp.float32,
                                      precision=lax.Precision.HIGHEST)


def _tc_bnmm(p, s, q, g, be, d, w):
    return pl.pallas_call(
        _bnmm_body,
        grid=(NBLK,),
        in_specs=[pl.BlockSpec((BLK, D), lambda i: (i, 0)),
                  pl.BlockSpec((1, D), lambda i: (0, 0)),
                  pl.BlockSpec((1, D), lambda i: (0, 0)),
                  pl.BlockSpec((1, D), lambda i: (0, 0)),
                  pl.BlockSpec((1, D), lambda i: (0, 0)),
                  pl.BlockSpec((BLK, 1), lambda i: (i, 0)),
                  pl.BlockSpec((D, D), lambda i: (0, 0))],
        out_specs=pl.BlockSpec((BLK, D), lambda i: (i, 0)),
        out_shape=jax.ShapeDtypeStruct((N, D), jnp.float32),
    )(p, s, q, g, be, d, w)


def _bn_body(p_ref, s_ref, q_ref, g_ref, be_ref, d_ref, o_ref):
    o_ref[...] = d_ref[...] * _bn_core(p_ref, s_ref, q_ref, g_ref, be_ref)


def _tc_bn(p, s, q, g, be, d):
    return pl.pallas_call(
        _bn_body,
        grid=(NBLK,),
        in_specs=[pl.BlockSpec((BLK, D), lambda i: (i, 0)),
                  pl.BlockSpec((1, D), lambda i: (0, 0)),
                  pl.BlockSpec((1, D), lambda i: (0, 0)),
                  pl.BlockSpec((1, D), lambda i: (0, 0)),
                  pl.BlockSpec((1, D), lambda i: (0, 0)),
                  pl.BlockSpec((BLK, 1), lambda i: (i, 0))],
        out_specs=pl.BlockSpec((BLK, D), lambda i: (i, 0)),
        out_shape=jax.ShapeDtypeStruct((N, D), jnp.float32),
    )(p, s, q, g, be, d)


def _final_body(a_ref, z_ref, d_ref, w_ref, b_ref, o_ref):
    u = d_ref[...] * (a_ref[0] + a_ref[1] + z_ref[...])
    h = jnp.dot(u, w_ref[...], preferred_element_type=jnp.float32,
                precision=lax.Precision.HIGHEST) + b_ref[...]
    m = jnp.max(h, axis=1, keepdims=True)
    sh = h - m
    o_ref[...] = sh - jnp.log(jnp.sum(jnp.exp(sh), axis=1, keepdims=True))


def _tc_final(a, z, d, w, b):
    return pl.pallas_call(
        _final_body,
        grid=(NBLK,),
        in_specs=[pl.BlockSpec((2, BLK, D), lambda i: (0, i, 0)),
                  pl.BlockSpec((BLK, D), lambda i: (i, 0)),
                  pl.BlockSpec((BLK, 1), lambda i: (i, 0)),
                  pl.BlockSpec((D, DOUT), lambda i: (0, 0)),
                  pl.BlockSpec((1, DOUT), lambda i: (0, 0))],
        out_specs=pl.BlockSpec((BLK, DOUT), lambda i: (i, 0)),
        out_shape=jax.ShapeDtypeStruct((N, DOUT), jnp.float32),
    )(a, z, d, w, b)


def kernel(x, edge_index, W0, b0, g0, be0, W1, b1, g1, be1, W2, b2):
    pad = E_PAD - E
    srcp = jnp.concatenate([edge_index[0], jnp.zeros((pad,), jnp.int32)])
    dstp = jnp.concatenate([edge_index[1], jnp.full((pad,), N, jnp.int32)])
    src_g = srcp.reshape(NW, CH, CK)
    dst_g = dstp.reshape(NW, CH, CK)
    dst_h = dstp.reshape(NW, TPW)
    zer = jnp.zeros((ZR, D), jnp.float32)

    hp = _sc_hist(dst_h)
    xw0 = _tc_mm(x, W0)
    dinv = _tc_prep(hp).reshape(NPAD, 1)[:N]

    z0 = _tc_scale(dinv, xw0)
    a0 = _sc_agg(z0, src_g, dst_g, zer)
    p1, s1, q1 = _tc_postagg(a0, z0, dinv, b0.reshape(1, D))
    z1 = _tc_bnmm(p1, s1, q1, g0.reshape(1, D), be0.reshape(1, D), dinv, W1)

    a1 = _sc_agg(z1, src_g, dst_g, zer)
    p2, s2, q2 = _tc_postagg(a1, z1, dinv, b1.reshape(1, D))
    z2 = _tc_bn(p2, s2, q2, g1.reshape(1, D), be1.reshape(1, D), dinv)

    a2 = _sc_agg(z2, src_g, dst_g, zer)
    return _tc_final(a2, z2, dinv, W2, b2.reshape(1, DOUT))


# speedup vs baseline: 8.9826x; 8.9826x over previous
import dataclasses
import functools

import jax
import jax.numpy as jnp
from jax import lax
from jax.experimental import pallas as pl
from jax.experimental.pallas import tpu as pltpu
from jax.experimental.pallas import tpu_sc as plsc

N = 10000
D = 128
DOUT = 40
E = 320000

NC = 2
NS = 16
NW = NC * NS
CK = 128
CH = 80
TPW = CH * CK
E_PAD = NW * TPW
NPAD = 10240
RPT = NPAD // NS
IG = 16

BLK = 1000
NBLK = N // BLK
EPS = 1e-5


def _sc_compiler_params():
    cp = pltpu.CompilerParams()
    if "needs_layout_passes" in pltpu.CompilerParams.__dataclass_fields__:
        cp = dataclasses.replace(cp, needs_layout_passes=False)
    return cp


@functools.cache
def _sc_hist_fn():
    mesh = plsc.VectorSubcoreMesh(core_axis_name="c", subcore_axis_name="s")

    @functools.partial(
        pl.kernel,
        out_type=jax.ShapeDtypeStruct((NW, NPAD), jnp.float32),
        mesh=mesh,
        compiler_params=_sc_compiler_params(),
        scratch_types=[
            pltpu.VMEM((TPW,), jnp.int32),
            pltpu.VMEM((NPAD,), jnp.float32),
        ],
    )
    def _sc_hist(dst_hbm, out_hbm, idxv, hist):
        c = lax.axis_index("c")
        s = lax.axis_index("s")
        wid = s * NC + c
        pltpu.sync_copy(dst_hbm.at[wid], idxv)
        zero = jnp.zeros((16,), jnp.float32)

        @pl.loop(0, NPAD, step=16)
        def _(i):
            hist[pl.ds(i, 16)] = zero

        ones = jnp.ones((16,), jnp.float32)

        @pl.loop(0, TPW, step=16)
        def _(i):
            idx = idxv[pl.ds(i, 16)]
            plsc.addupdate_scatter(hist, [idx], ones)

        pltpu.sync_copy(hist, out_hbm.at[wid])

    return _sc_hist


@functools.cache
def _sc_agg_fn():
    mesh = plsc.VectorSubcoreMesh(core_axis_name="c", subcore_axis_name="s")

    @functools.partial(
        pl.kernel,
        out_type=jax.ShapeDtypeStruct((NC, NPAD, D), jnp.float32),
        mesh=mesh,
        scratch_types=[
            pltpu.VMEM((IG, CK), jnp.int32),
            pltpu.VMEM((IG, CK), jnp.int32),
            pltpu.VMEM((CK, D), jnp.float32),
            pltpu.VMEM((CK, D), jnp.float32),
            pltpu.VMEM_SHARED((NPAD, D), jnp.float32),
            pltpu.SemaphoreType.DMA,
            pltpu.SemaphoreType.DMA,
        ],
    )
    def _sc_agg(z_hbm, src_hbm, dst_hbm, zer_hbm, out_hbm,
                srcv, dstv, rows_a, rows_b, acc, sem_a, sem_b):
        c = lax.axis_index("c")
        s = lax.axis_index("s")
        wid = s * NC + c

        pltpu.sync_copy(zer_hbm, rows_a)

        @pl.loop(0, RPT, step=CK)
        def _(i):
            pltpu.sync_copy(rows_a, acc.at[pl.ds(s * RPT + i, CK)])

        plsc.subcore_barrier()

        @pl.loop(0, CH, step=IG)
        def _(g):
            pltpu.sync_copy(src_hbm.at[wid, pl.ds(g, IG)], srcv)
            pltpu.sync_copy(dst_hbm.at[wid, pl.ds(g, IG)], dstv)
            pltpu.make_async_copy(z_hbm.at[srcv.at[0]], rows_a, sem_a).start()

            @pl.loop(0, IG, step=2)
            def _(j):
                pltpu.make_async_copy(z_hbm.at[srcv.at[j]], rows_a, sem_a).wait()
                pltpu.make_async_copy(z_hbm.at[srcv.at[j + 1]], rows_b, sem_b).start()
                pltpu.sync_copy(rows_a, acc.at[dstv.at[j]], add=True)
                pltpu.make_async_copy(z_hbm.at[srcv.at[j + 1]], rows_b, sem_b).wait()

                @pl.when(j + 2 < IG)
                def _():
                    pltpu.make_async_copy(z_hbm.at[srcv.at[j + 2]], rows_a, sem_a).start()

                pltpu.sync_copy(rows_b, acc.at[dstv.at[j + 1]], add=True)

        plsc.subcore_barrier()
        pltpu.sync_copy(acc.at[pl.ds(s * RPT, RPT)],
                        out_hbm.at[c, pl.ds(s * RPT, RPT)])

    return _sc_agg


def _mm_body(x_ref, w_ref, o_ref):
    o_ref[...] = jnp.dot(x_ref[...], w_ref[...],
                         preferred_element_type=jnp.float32,
                         precision=lax.Precision.HIGHEST)


def _tc_mm(x, w):
    m, k = x.shape
    n = w.shape[1]
    return pl.pallas_call(
        _mm_body,
        grid=(m // BLK,),
        in_specs=[pl.BlockSpec((BLK, k), lambda i: (i, 0)),
                  pl.BlockSpec((k, n), lambda i: (0, 0))],
        out_specs=pl.BlockSpec((BLK, n), lambda i: (i, 0)),
        out_shape=jax.ShapeDtypeStruct((m, n), jnp.float32),
    )(x, w)


def _prep_body(hp_ref, o_ref):
    o_ref[...] = lax.rsqrt(1.0 + jnp.sum(hp_ref[...], axis=0, keepdims=True))


def _tc_prep(hp):
    return pl.pallas_call(
        _prep_body,
        out_shape=jax.ShapeDtypeStruct((1, NPAD), jnp.float32),
    )(hp)


def _scale_body(d_ref, x_ref, o_ref):
    o_ref[...] = d_ref[...] * x_ref[...]


def _tc_scale(d, x):
    return pl.pallas_call(
        _scale_body,
        grid=(NBLK,),
        in_specs=[pl.BlockSpec((BLK, 1), lambda i: (i, 0)),
                  pl.BlockSpec((BLK, D), lambda i: (i, 0))],
        out_specs=pl.BlockSpec((BLK, D), lambda i: (i, 0)),
        out_shape=jax.ShapeDtypeStruct((N, D), jnp.float32),
    )(d, x)


def _postagg_body(a_ref, z_ref, d_ref, b_ref, p_ref, s_ref, q_ref):
    agg = a_ref[0] + a_ref[1]
    p = d_ref[...] * (agg + z_ref[...]) + b_ref[...]
    p_ref[...] = p

    @pl.when(pl.program_id(0) == 0)
    def _():
        s_ref[...] = jnp.zeros_like(s_ref)
        q_ref[...] = jnp.zeros_like(q_ref)

    s_ref[...] += jnp.sum(p, axis=0, keepdims=True)
    q_ref[...] += jnp.sum(p * p, axis=0, keepdims=True)


def _tc_postagg(a, z, d, b):
    return pl.pallas_call(
        _postagg_body,
        grid=(NBLK,),
        in_specs=[pl.BlockSpec((2, BLK, D), lambda i: (0, i, 0)),
                  pl.BlockSpec((BLK, D), lambda i: (i, 0)),
                  pl.BlockSpec((BLK, 1), lambda i: (i, 0)),
                  pl.BlockSpec((1, D), lambda i: (0, 0))],
        out_specs=[pl.BlockSpec((BLK, D), lambda i: (i, 0)),
                   pl.BlockSpec((1, D), lambda i: (0, 0)),
                   pl.BlockSpec((1, D), lambda i: (0, 0))],
        out_shape=[jax.ShapeDtypeStruct((N, D), jnp.float32),
                   jax.ShapeDtypeStruct((1, D), jnp.float32),
                   jax.ShapeDtypeStruct((1, D), jnp.float32)],
    )(a, z, d, b)


def _bn_core(p_ref, s_ref, q_ref, g_ref, be_ref):
    mu = s_ref[...] * (1.0 / N)
    var = q_ref[...] * (1.0 / N) - mu * mu
    a = g_ref[...] * lax.rsqrt(var + EPS)
    return jnp.maximum((p_ref[...] - mu) * a + be_ref[...], 0.0)


def _bnmm_body(p_ref, s_ref, q_ref, g_ref, be_ref, d_ref, w_ref, o_ref):
    h = _bn_core(p_ref, s_ref, q_ref, g_ref, be_ref)
    o_ref[...] = d_ref[...] * jnp.dot(h, w_ref[...],
                                      preferred_element_type=jnp.float32,
                                      precision=lax.Precision.HIGHEST)


def _tc_bnmm(p, s, q, g, be, d, w):
    return pl.pallas_call(
        _bnmm_body,
        grid=(NBLK,),
        in_specs=[pl.BlockSpec((BLK, D), lambda i: (i, 0)),
                  pl.BlockSpec((1, D), lambda i: (0, 0)),
                  pl.BlockSpec((1, D), lambda i: (0, 0)),
                  pl.BlockSpec((1, D), lambda i: (0, 0)),
                  pl.BlockSpec((1, D), lambda i: (0, 0)),
                  pl.BlockSpec((BLK, 1), lambda i: (i, 0)),
                  pl.BlockSpec((D, D), lambda i: (0, 0))],
        out_specs=pl.BlockSpec((BLK, D), lambda i: (i, 0)),
        out_shape=jax.ShapeDtypeStruct((N, D), jnp.float32),
    )(p, s, q, g, be, d, w)


def _bn_body(p_ref, s_ref, q_ref, g_ref, be_ref, d_ref, o_ref):
    o_ref[...] = d_ref[...] * _bn_core(p_ref, s_ref, q_ref, g_ref, be_ref)


def _tc_bn(p, s, q, g, be, d):
    return pl.pallas_call(
        _bn_body,
        grid=(NBLK,),
        in_specs=[pl.BlockSpec((BLK, D), lambda i: (i, 0)),
                  pl.BlockSpec((1, D), lambda i: (0, 0)),
                  pl.BlockSpec((1, D), lambda i: (0, 0)),
                  pl.BlockSpec((1, D), lambda i: (0, 0)),
                  pl.BlockSpec((1, D), lambda i: (0, 0)),
                  pl.BlockSpec((BLK, 1), lambda i: (i, 0))],
        out_specs=pl.BlockSpec((BLK, D), lambda i: (i, 0)),
        out_shape=jax.ShapeDtypeStruct((N, D), jnp.float32),
    )(p, s, q, g, be, d)


def _final_body(a_ref, z_ref, d_ref, w_ref, b_ref, o_ref):
    u = d_ref[...] * (a_ref[0] + a_ref[1] + z_ref[...])
    h = jnp.dot(u, w_ref[...], preferred_element_type=jnp.float32,
                precision=lax.Precision.HIGHEST) + b_ref[...]
    m = jnp.max(h, axis=1, keepdims=True)
    sh = h - m
    o_ref[...] = sh - jnp.log(jnp.sum(jnp.exp(sh), axis=1, keepdims=True))


def _tc_final(a, z, d, w, b):
    return pl.pallas_call(
        _final_body,
        grid=(NBLK,),
        in_specs=[pl.BlockSpec((2, BLK, D), lambda i: (0, i, 0)),
                  pl.BlockSpec((BLK, D), lambda i: (i, 0)),
                  pl.BlockSpec((BLK, 1), lambda i: (i, 0)),
                  pl.BlockSpec((D, DOUT), lambda i: (0, 0)),
                  pl.BlockSpec((1, DOUT), lambda i: (0, 0))],
        out_specs=pl.BlockSpec((BLK, DOUT), lambda i: (i, 0)),
        out_shape=jax.ShapeDtypeStruct((N, DOUT), jnp.float32),
    )(a, z, d, w, b)


def kernel(x, edge_index, W0, b0, g0, be0, W1, b1, g1, be1, W2, b2):
    pad = E_PAD - E
    srcp = jnp.concatenate([edge_index[0], jnp.zeros((pad,), jnp.int32)])
    dstp = jnp.concatenate([edge_index[1], jnp.full((pad,), N, jnp.int32)])
    src_g = srcp.reshape(NW, CH, CK)
    dst_g = dstp.reshape(NW, CH, CK)
    dst_h = dstp.reshape(NW, TPW)
    zer = jnp.zeros((CK, D), jnp.float32)

    sc_hist = _sc_hist_fn()
    sc_agg = _sc_agg_fn()

    hp = sc_hist(dst_h)
    xw0 = _tc_mm(x, W0)
    dinv = _tc_prep(hp).reshape(NPAD, 1)[:N]

    z0 = _tc_scale(dinv, xw0)
    a0 = sc_agg(z0, src_g, dst_g, zer)
    p1, s1, q1 = _tc_postagg(a0, z0, dinv, b0.reshape(1, D))
    z1 = _tc_bnmm(p1, s1, q1, g0.reshape(1, D), be0.reshape(1, D), dinv, W1)

    a1 = sc_agg(z1, src_g, dst_g, zer)
    p2, s2, q2 = _tc_postagg(a1, z1, dinv, b1.reshape(1, D))
    z2 = _tc_bn(p2, s2, q2, g1.reshape(1, D), be1.reshape(1, D), dinv)

    a2 = sc_agg(z2, src_g, dst_g, zer)
    return _tc_final(a2, z2, dinv, W2, b2.reshape(1, DOUT))


# speedup vs baseline: 23.9390x; 2.6651x over previous
import dataclasses
import functools

import jax
import jax.numpy as jnp
from jax import lax
from jax.experimental import pallas as pl
from jax.experimental.pallas import tpu as pltpu
from jax.experimental.pallas import tpu_sc as plsc

N = 10000
D = 128
DOUT = 40
E = 320000

NC = 2
NS = 16
NW = NC * NS
CK = 128
CH = 80
TPW = CH * CK
E_PAD = NW * TPW
TOT_CH = NW * CH
NPAD = 10240
RPT = NPAD // NS
IG = 40
HK = CK // 2

BLK = 2000
NBLK = N // BLK
EPS = 1e-5


def _sc_compiler_params():
    cp = pltpu.CompilerParams()
    if "needs_layout_passes" in pltpu.CompilerParams.__dataclass_fields__:
        cp = dataclasses.replace(cp, needs_layout_passes=False)
    return cp


@functools.cache
def _sc_hist_fn():
    mesh = plsc.VectorSubcoreMesh(core_axis_name="c", subcore_axis_name="s")

    @functools.partial(
        pl.kernel,
        out_type=jax.ShapeDtypeStruct((NW, NPAD), jnp.float32),
        mesh=mesh,
        compiler_params=_sc_compiler_params(),
        scratch_types=[
            pltpu.VMEM((TPW,), jnp.int32),
            pltpu.VMEM((NPAD,), jnp.float32),
        ],
    )
    def _sc_hist(dst_hbm, out_hbm, idxv, hist):
        c = lax.axis_index("c")
        s = lax.axis_index("s")
        wid = s * NC + c
        pltpu.sync_copy(dst_hbm.at[wid], idxv)
        zero = jnp.zeros((16,), jnp.float32)

        @pl.loop(0, NPAD, step=16)
        def _(i):
            hist[pl.ds(i, 16)] = zero

        ones = jnp.ones((16,), jnp.float32)

        @pl.loop(0, TPW, step=16)
        def _(i):
            idx = idxv[pl.ds(i, 16)]
            plsc.addupdate_scatter(hist, [idx], ones)

        pltpu.sync_copy(hist, out_hbm.at[wid])

    return _sc_hist


@functools.cache
def _sc_agg_fn():
    mesh = plsc.VectorSubcoreMesh(core_axis_name="c", subcore_axis_name="s")

    @functools.partial(
        pl.kernel,
        out_type=jax.ShapeDtypeStruct((NC, NPAD, D), jnp.float32),
        mesh=mesh,
        scratch_types=[
            pltpu.VMEM((IG, CK), jnp.int32),
            pltpu.VMEM((IG, CK), jnp.int32),
            pltpu.VMEM((CK, D), jnp.float32),
            pltpu.VMEM((CK, D), jnp.float32),
            pltpu.VMEM_SHARED((NPAD, D), jnp.float32),
            pltpu.SemaphoreType.DMA,
            pltpu.SemaphoreType.DMA,
            pltpu.SemaphoreType.DMA,
            pltpu.SemaphoreType.DMA,
        ],
    )
    def _sc_agg(z_hbm, src_hbm, dst_hbm, zer_hbm, out_hbm,
                srcv, dstv, rows_a, rows_b, acc, sem_a, sem_b, sem_a2, sem_b2):

        def gather_start(j, buf, s1, s2):
            pltpu.make_async_copy(z_hbm.at[srcv.at[j, pl.ds(0, HK)]],
                                  buf.at[pl.ds(0, HK)], s1).start()
            pltpu.make_async_copy(z_hbm.at[srcv.at[j, pl.ds(HK, HK)]],
                                  buf.at[pl.ds(HK, HK)], s2).start()

        def gather_wait(j, buf, s1, s2):
            pltpu.make_async_copy(z_hbm.at[srcv.at[j, pl.ds(0, HK)]],
                                  buf.at[pl.ds(0, HK)], s1).wait()
            pltpu.make_async_copy(z_hbm.at[srcv.at[j, pl.ds(HK, HK)]],
                                  buf.at[pl.ds(HK, HK)], s2).wait()
        c = lax.axis_index("c")
        s = lax.axis_index("s")
        base = (s * NC + c) * CH
        nch = CH

        pltpu.sync_copy(src_hbm.at[pl.ds(base, IG)], srcv)
        pltpu.sync_copy(dst_hbm.at[pl.ds(base, IG)], dstv)
        gather_start(0, rows_a, sem_a, sem_a2)

        pltpu.sync_copy(zer_hbm, rows_b)

        @pl.loop(0, RPT, step=16)
        def _(i):
            pltpu.sync_copy(rows_b.at[pl.ds(0, 16)], acc.at[pl.ds(s * RPT + i, 16)])

        plsc.subcore_barrier()

        @pl.loop(0, nch, step=IG)
        def _(g):
            @pl.when(g > 0)
            def _():
                pltpu.sync_copy(src_hbm.at[pl.ds(base + g, IG)], srcv)
                pltpu.sync_copy(dst_hbm.at[pl.ds(base + g, IG)], dstv)
                gather_start(0, rows_a, sem_a, sem_a2)

            @pl.loop(0, IG, step=2)
            def _(j):
                gather_wait(j, rows_a, sem_a, sem_a2)
                gather_start(j + 1, rows_b, sem_b, sem_b2)
                pltpu.sync_copy(rows_a, acc.at[dstv.at[j]], add=True)
                gather_wait(j + 1, rows_b, sem_b, sem_b2)

                @pl.when(j + 2 < IG)
                def _():
                    gather_start(j + 2, rows_a, sem_a, sem_a2)

                pltpu.sync_copy(rows_b, acc.at[dstv.at[j + 1]], add=True)

        plsc.subcore_barrier()
        pltpu.sync_copy(acc.at[pl.ds(s * RPT, RPT)],
                        out_hbm.at[c, pl.ds(s * RPT, RPT)])

    return _sc_agg


def _mmscale_body(d_ref, x_ref, w_ref, o_ref):
    o_ref[...] = d_ref[...] * jnp.dot(x_ref[...], w_ref[...],
                                      preferred_element_type=jnp.float32,
                                      precision=lax.Precision.HIGHEST)


def _tc_mmscale(d, x, w):
    m, k = x.shape
    n = w.shape[1]
    return pl.pallas_call(
        _mmscale_body,
        grid=(m // BLK,),
        in_specs=[pl.BlockSpec((BLK, 1), lambda i: (i, 0)),
                  pl.BlockSpec((BLK, k), lambda i: (i, 0)),
                  pl.BlockSpec((k, n), lambda i: (0, 0))],
        out_specs=pl.BlockSpec((BLK, n), lambda i: (i, 0)),
        out_shape=jax.ShapeDtypeStruct((m, n), jnp.float32),
    )(d, x, w)


def _prep_body(hp_ref, o_ref):
    o_ref[...] = lax.rsqrt(1.0 + jnp.sum(hp_ref[...], axis=0, keepdims=True))


def _tc_prep(hp):
    return pl.pallas_call(
        _prep_body,
        out_shape=jax.ShapeDtypeStruct((1, NPAD), jnp.float32),
    )(hp)


def _scale_body(d_ref, x_ref, o_ref):
    o_ref[...] = d_ref[...] * x_ref[...]


def _tc_scale(d, x):
    return pl.pallas_call(
        _scale_body,
        grid=(NBLK,),
        in_specs=[pl.BlockSpec((BLK, 1), lambda i: (i, 0)),
                  pl.BlockSpec((BLK, D), lambda i: (i, 0))],
        out_specs=pl.BlockSpec((BLK, D), lambda i: (i, 0)),
        out_shape=jax.ShapeDtypeStruct((N, D), jnp.float32),
    )(d, x)


def _postagg_body(a_ref, z_ref, d_ref, b_ref, p_ref, s_ref, q_ref):
    agg = a_ref[0] + a_ref[1]
    p = d_ref[...] * (agg + z_ref[...]) + b_ref[...]
    p_ref[...] = p

    @pl.when(pl.program_id(0) == 0)
    def _():
        s_ref[...] = jnp.zeros_like(s_ref)
        q_ref[...] = jnp.zeros_like(q_ref)

    s_ref[...] += jnp.sum(p, axis=0, keepdims=True)
    q_ref[...] += jnp.sum(p * p, axis=0, keepdims=True)


def _tc_postagg(a, z, d, b):
    return pl.pallas_call(
        _postagg_body,
        grid=(NBLK,),
        in_specs=[pl.BlockSpec((2, BLK, D), lambda i: (0, i, 0)),
                  pl.BlockSpec((BLK, D), lambda i: (i, 0)),
                  pl.BlockSpec((BLK, 1), lambda i: (i, 0)),
                  pl.BlockSpec((1, D), lambda i: (0, 0))],
        out_specs=[pl.BlockSpec((BLK, D), lambda i: (i, 0)),
                   pl.BlockSpec((1, D), lambda i: (0, 0)),
                   pl.BlockSpec((1, D), lambda i: (0, 0))],
        out_shape=[jax.ShapeDtypeStruct((N, D), jnp.float32),
                   jax.ShapeDtypeStruct((1, D), jnp.float32),
                   jax.ShapeDtypeStruct((1, D), jnp.float32)],
    )(a, z, d, b)


def _bn_core(p_ref, s_ref, q_ref, g_ref, be_ref):
    mu = s_ref[...] * (1.0 / N)
    var = q_ref[...] * (1.0 / N) - mu * mu
    a = g_ref[...] * lax.rsqrt(var + EPS)
    return jnp.maximum((p_ref[...] - mu) * a + be_ref[...], 0.0)


def _bnmm_body(p_ref, s_ref, q_ref, g_ref, be_ref, d_ref, w_ref, o_ref):
    h = _bn_core(p_ref, s_ref, q_ref, g_ref, be_ref)
    o_ref[...] = d_ref[...] * jnp.dot(h, w_ref[...],
                                      preferred_element_type=jnp.float32,
                                      precision=lax.Precision.HIGHEST)


def _tc_bnmm(p, s, q, g, be, d, w):
    return pl.pallas_call(
        _bnmm_body,
        grid=(NBLK,),
        in_specs=[pl.BlockSpec((BLK, D), lambda i: (i, 0)),
                  pl.BlockSpec((1, D), lambda i: (0, 0)),
                  pl.BlockSpec((1, D), lambda i: (0, 0)),
                  pl.BlockSpec((1, D), lambda i: (0, 0)),
                  pl.BlockSpec((1, D), lambda i: (0, 0)),
                  pl.BlockSpec((BLK, 1), lambda i: (i, 0)),
                  pl.BlockSpec((D, D), lambda i: (0, 0))],
        out_specs=pl.BlockSpec((BLK, D), lambda i: (i, 0)),
        out_shape=jax.ShapeDtypeStruct((N, D), jnp.float32),
    )(p, s, q, g, be, d, w)


def _bn_body(p_ref, s_ref, q_ref, g_ref, be_ref, d_ref, o_ref):
    o_ref[...] = d_ref[...] * _bn_core(p_ref, s_ref, q_ref, g_ref, be_ref)


def _tc_bn(p, s, q, g, be, d):
    return pl.pallas_call(
        _bn_body,
        grid=(NBLK,),
        in_specs=[pl.BlockSpec((BLK, D), lambda i: (i, 0)),
                  pl.BlockSpec((1, D), lambda i: (0, 0)),
                  pl.BlockSpec((1, D), lambda i: (0, 0)),
                  pl.BlockSpec((1, D), lambda i: (0, 0)),
                  pl.BlockSpec((1, D), lambda i: (0, 0)),
                  pl.BlockSpec((BLK, 1), lambda i: (i, 0))],
        out_specs=pl.BlockSpec((BLK, D), lambda i: (i, 0)),
        out_shape=jax.ShapeDtypeStruct((N, D), jnp.float32),
    )(p, s, q, g, be, d)


def _final_body(a_ref, z_ref, d_ref, w_ref, b_ref, o_ref):
    u = d_ref[...] * (a_ref[0] + a_ref[1] + z_ref[...])
    h = jnp.dot(u, w_ref[...], preferred_element_type=jnp.float32,
                precision=lax.Precision.HIGHEST) + b_ref[...]
    m = jnp.max(h, axis=1, keepdims=True)
    sh = h - m
    o_ref[...] = sh - jnp.log(jnp.sum(jnp.exp(sh), axis=1, keepdims=True))


def _tc_final(a, z, d, w, b):
    return pl.pallas_call(
        _final_body,
        grid=(NBLK,),
        in_specs=[pl.BlockSpec((2, BLK, D), lambda i: (0, i, 0)),
                  pl.BlockSpec((BLK, D), lambda i: (i, 0)),
                  pl.BlockSpec((BLK, 1), lambda i: (i, 0)),
                  pl.BlockSpec((D, DOUT), lambda i: (0, 0)),
                  pl.BlockSpec((1, DOUT), lambda i: (0, 0))],
        out_specs=pl.BlockSpec((BLK, DOUT), lambda i: (i, 0)),
        out_shape=jax.ShapeDtypeStruct((N, DOUT), jnp.float32),
    )(a, z, d, w, b)


def kernel(x, edge_index, W0, b0, g0, be0, W1, b1, g1, be1, W2, b2):
    pad = E_PAD - E
    junk = N + jnp.arange(pad, dtype=jnp.int32) % (NPAD - N)
    jsrc = jnp.arange(pad, dtype=jnp.int32) % N
    srcp = jnp.concatenate([edge_index[0], jsrc])
    dstp = jnp.concatenate([edge_index[1], junk])
    src_g = srcp.reshape(TOT_CH, CK)
    dst_g = dstp.reshape(TOT_CH, CK)
    dst_h = dstp.reshape(NW, TPW)
    zer = jnp.zeros((CK, D), jnp.float32)

    sc_hist = _sc_hist_fn()
    sc_agg = _sc_agg_fn()

    hp = sc_hist(dst_h)
    dinv = _tc_prep(hp).reshape(NPAD, 1)[:N]

    z0 = _tc_mmscale(dinv, x, W0)
    a0 = sc_agg(z0, src_g, dst_g, zer)
    p1, s1, q1 = _tc_postagg(a0, z0, dinv, b0.reshape(1, D))
    z1 = _tc_bnmm(p1, s1, q1, g0.reshape(1, D), be0.reshape(1, D), dinv, W1)

    a1 = sc_agg(z1, src_g, dst_g, zer)
    p2, s2, q2 = _tc_postagg(a1, z1, dinv, b1.reshape(1, D))
    z2 = _tc_bn(p2, s2, q2, g1.reshape(1, D), be1.reshape(1, D), dinv)

    a2 = sc_agg(z2, src_g, dst_g, zer)
    return _tc_final(a2, z2, dinv, W2, b2.reshape(1, DOUT))
